# Initial kernel scaffold; baseline (speedup 1.0000x reference)
import jax
import jax.numpy as jnp
from jax.experimental import pallas as pl


def kernel(sequence, weight):
    raise NotImplementedError("write your pallas kernel here")


# speedup vs baseline: 1.7957x; 1.7957x over previous
import functools

import jax
import jax.numpy as jnp
from jax import lax
from jax.experimental import pallas as pl
from jax.experimental.pallas import tpu as pltpu
from jax.experimental.pallas import tpu_sc as plsc

NUM_CORES = 2
NUM_SUBCORES = 16
NW = NUM_CORES * NUM_SUBCORES

SEQ_ROWS = 16384
SEQ_COLS = 50
D = 64
B = SEQ_ROWS * SEQ_COLS
B_PER_W = B // NW
CHUNK = 512
N_CHUNKS = B_PER_W // CHUNK


def _body(seq_hbm, table_hbm, out_hbm, idx_v, rows_v, sem):
    wid = lax.axis_index("s") * NUM_CORES + lax.axis_index("c")
    base = wid * B_PER_W

    def step(i, carry):
        off = base + i * CHUNK
        pltpu.sync_copy(seq_hbm.at[pl.ds(off, CHUNK)], idx_v)
        pltpu.async_copy(table_hbm.at[idx_v], rows_v, sem).wait()
        pltpu.sync_copy(rows_v, out_hbm.at[pl.ds(off, CHUNK)])
        return carry

    lax.fori_loop(0, N_CHUNKS, step, 0)


@jax.jit
def _embed(seq_flat, weight):
    mesh = plsc.VectorSubcoreMesh(
        core_axis_name="c", subcore_axis_name="s",
        num_cores=NUM_CORES, num_subcores=NUM_SUBCORES)
    k = pl.kernel(
        _body,
        out_type=jax.ShapeDtypeStruct((B, D), jnp.float32),
        mesh=mesh,
        scratch_types=[
            pltpu.VMEM((CHUNK,), jnp.int32),
            pltpu.VMEM((CHUNK, D), jnp.float32),
            pltpu.SemaphoreType.DMA,
        ],
        compiler_params=pltpu.CompilerParams(use_tc_tiling_on_sc=False),
    )
    return k(seq_flat, weight)


def kernel(sequence, weight):
    out = _embed(sequence.reshape(-1), weight)
    return out.reshape(SEQ_ROWS, SEQ_COLS, D)


# speedup vs baseline: 1.8726x; 1.0428x over previous
import functools

import jax
import jax.numpy as jnp
from jax import lax
from jax.experimental import pallas as pl
from jax.experimental.pallas import tpu as pltpu
from jax.experimental.pallas import tpu_sc as plsc

NUM_CORES = 2
NUM_SUBCORES = 16
NW = NUM_CORES * NUM_SUBCORES

SEQ_ROWS = 16384
SEQ_COLS = 50
D = 64
B = SEQ_ROWS * SEQ_COLS
B_PER_W = B // NW
CHUNK = 512
N_CHUNKS = B_PER_W // CHUNK


def _body(seq_hbm, table_hbm, out_hbm, idx_v, rows_v, gsem0, gsem1, ssem0,
          ssem1):
    wid = lax.axis_index("s") * NUM_CORES + lax.axis_index("c")
    base = wid * B_PER_W

    pltpu.sync_copy(seq_hbm.at[pl.ds(base, B_PER_W)], idx_v)

    def g_copy(i, b, sem):
        return pltpu.make_async_copy(
            table_hbm.at[idx_v.at[pl.ds(i * CHUNK, CHUNK)]],
            rows_v.at[b], sem)

    def s_copy(i, b, sem):
        return pltpu.make_async_copy(
            rows_v.at[b], out_hbm.at[pl.ds(base + i * CHUNK, CHUNK)], sem)

    N, P = N_CHUNKS, N_CHUNKS // 2

    g_copy(0, 0, gsem0).start()
    g_copy(0, 0, gsem0).wait()
    s_copy(0, 0, ssem0).start()
    g_copy(1, 1, gsem1).start()
    g_copy(1, 1, gsem1).wait()
    s_copy(1, 1, ssem1).start()
    s_copy(0, 0, ssem0).wait()
    g_copy(2, 0, gsem0).start()

    def pair(p, carry):
        i0 = 2 * p
        g_copy(i0, 0, gsem0).wait()
        s_copy(i0, 0, ssem0).start()
        s_copy(i0 - 1, 1, ssem1).wait()
        g_copy(i0 + 1, 1, gsem1).start()
        g_copy(i0 + 1, 1, gsem1).wait()
        s_copy(i0 + 1, 1, ssem1).start()
        s_copy(i0, 0, ssem0).wait()
        g_copy(i0 + 2, 0, gsem0).start()
        return carry

    lax.fori_loop(1, P - 1, pair, 0)

    i0 = N - 2
    g_copy(i0, 0, gsem0).wait()
    s_copy(i0, 0, ssem0).start()
    s_copy(i0 - 1, 1, ssem1).wait()
    g_copy(i0 + 1, 1, gsem1).start()
    g_copy(i0 + 1, 1, gsem1).wait()
    s_copy(i0 + 1, 1, ssem1).start()
    s_copy(i0, 0, ssem0).wait()
    s_copy(i0 + 1, 1, ssem1).wait()


@jax.jit
def _embed(seq_flat, weight):
    mesh = plsc.VectorSubcoreMesh(
        core_axis_name="c", subcore_axis_name="s",
        num_cores=NUM_CORES, num_subcores=NUM_SUBCORES)
    k = pl.kernel(
        _body,
        out_type=jax.ShapeDtypeStruct((B, D), jnp.float32),
        mesh=mesh,
        scratch_types=[
            pltpu.VMEM((B_PER_W,), jnp.int32),
            pltpu.VMEM((2, CHUNK, D), jnp.float32),
            pltpu.SemaphoreType.DMA,
            pltpu.SemaphoreType.DMA,
            pltpu.SemaphoreType.DMA,
            pltpu.SemaphoreType.DMA,
        ],
        compiler_params=pltpu.CompilerParams(use_tc_tiling_on_sc=False),
    )
    return k(seq_flat, weight)


def kernel(sequence, weight):
    out = _embed(sequence.reshape(-1), weight)
    return out.reshape(SEQ_ROWS, SEQ_COLS, D)


# speedup vs baseline: 1.8735x; 1.0005x over previous
import functools

import jax
import jax.numpy as jnp
from jax import lax
from jax.experimental import pallas as pl
from jax.experimental.pallas import tpu as pltpu
from jax.experimental.pallas import tpu_sc as plsc

NUM_CORES = 2
NUM_SUBCORES = 16
NW = NUM_CORES * NUM_SUBCORES

SEQ_ROWS = 16384
SEQ_COLS = 50
D = 64
V = 1000000
B = SEQ_ROWS * SEQ_COLS
B_PER_W = B // NW
CHUNK = 512
N_CHUNKS = B_PER_W // CHUNK


def _body(seq_hbm, table_hbm, out_hbm, idx_v, rows_v, gsem0, gsem1, ssem0,
          ssem1):
    wid = lax.axis_index("s") * NUM_CORES + lax.axis_index("c")
    base = wid * B_PER_W

    pltpu.sync_copy(seq_hbm.at[pl.ds(base, B_PER_W)], idx_v)

    def g_copy(i, b, sem):
        return pltpu.make_async_copy(
            table_hbm.at[idx_v.at[pl.ds(i * CHUNK, CHUNK)]],
            rows_v.at[b], sem)

    def s_copy(i, b, sem):
        return pltpu.make_async_copy(
            rows_v.at[b], out_hbm.at[pl.ds(base + i * CHUNK, CHUNK)], sem)

    N, P = N_CHUNKS, N_CHUNKS // 2

    g_copy(0, 0, gsem0).start()
    g_copy(0, 0, gsem0).wait()
    s_copy(0, 0, ssem0).start()
    g_copy(1, 1, gsem1).start()
    g_copy(1, 1, gsem1).wait()
    s_copy(1, 1, ssem1).start()
    s_copy(0, 0, ssem0).wait()
    g_copy(2, 0, gsem0).start()

    def pair(p, carry):
        i0 = 2 * p
        g_copy(i0, 0, gsem0).wait()
        s_copy(i0, 0, ssem0).start()
        s_copy(i0 - 1, 1, ssem1).wait()
        g_copy(i0 + 1, 1, gsem1).start()
        g_copy(i0 + 1, 1, gsem1).wait()
        s_copy(i0 + 1, 1, ssem1).start()
        s_copy(i0, 0, ssem0).wait()
        g_copy(i0 + 2, 0, gsem0).start()
        return carry

    lax.fori_loop(1, P - 1, pair, 0)

    i0 = N - 2
    g_copy(i0, 0, gsem0).wait()
    s_copy(i0, 0, ssem0).start()
    s_copy(i0 - 1, 1, ssem1).wait()
    g_copy(i0 + 1, 1, gsem1).start()
    g_copy(i0 + 1, 1, gsem1).wait()
    s_copy(i0 + 1, 1, ssem1).start()
    s_copy(i0, 0, ssem0).wait()
    s_copy(i0 + 1, 1, ssem1).wait()


@jax.jit
def _embed(sequence, weight):
    mesh = plsc.VectorSubcoreMesh(
        core_axis_name="c", subcore_axis_name="s",
        num_cores=NUM_CORES, num_subcores=NUM_SUBCORES)

    gather_k = pl.kernel(
        _body,
        out_type=jax.ShapeDtypeStruct((B, D), jnp.float32),
        mesh=mesh,
        scratch_types=[
            pltpu.VMEM((B_PER_W,), jnp.int32),
            pltpu.VMEM((2, CHUNK, D), jnp.float32),
            pltpu.SemaphoreType.DMA,
            pltpu.SemaphoreType.DMA,
            pltpu.SemaphoreType.DMA,
            pltpu.SemaphoreType.DMA,
        ],
        compiler_params=pltpu.CompilerParams(use_tc_tiling_on_sc=False),
    )
    return gather_k(sequence.reshape(-1), weight)


def kernel(sequence, weight):
    out = _embed(sequence, weight)
    return out.reshape(SEQ_ROWS, SEQ_COLS, D)


# speedup vs baseline: 1.9549x; 1.0434x over previous
import functools

import jax
import jax.numpy as jnp
from jax import lax
from jax.experimental import pallas as pl
from jax.experimental.pallas import tpu as pltpu
from jax.experimental.pallas import tpu_sc as plsc

NUM_CORES = 2
NUM_SUBCORES = 16
NW = NUM_CORES * NUM_SUBCORES

SEQ_ROWS = 16384
SEQ_COLS = 50
D = 64
V = 1000000
B = SEQ_ROWS * SEQ_COLS
B_PER_W = B // NW
CHUNK = 512
N_CHUNKS = B_PER_W // CHUNK


def _body(seq_hbm, table_hbm, out_hbm, idx_v, rows_v, gsem0, gsem1, ssem0,
          ssem1):
    wid = lax.axis_index("s") * NUM_CORES + lax.axis_index("c")
    base = wid * B_PER_W

    pltpu.sync_copy(seq_hbm.at[pl.ds(base, B_PER_W)], idx_v)

    def g_copy(i, b, sem):
        return pltpu.make_async_copy(
            table_hbm.at[idx_v.at[pl.ds(i * CHUNK, CHUNK)]],
            rows_v.at[b], sem)

    def s_copy(i, b, sem):
        return pltpu.make_async_copy(
            rows_v.at[b], out_hbm.at[pl.ds(base + i * CHUNK, CHUNK)], sem)

    N, P = N_CHUNKS, N_CHUNKS // 2

    g_copy(0, 0, gsem0).start()
    g_copy(0, 0, gsem0).wait()
    s_copy(0, 0, ssem0).start()
    g_copy(1, 1, gsem1).start()
    g_copy(1, 1, gsem1).wait()
    s_copy(1, 1, ssem1).start()
    s_copy(0, 0, ssem0).wait()
    g_copy(2, 0, gsem0).start()

    def pair(p, carry):
        i0 = 2 * p
        g_copy(i0, 0, gsem0).wait()
        s_copy(i0, 0, ssem0).start()
        s_copy(i0 - 1, 1, ssem1).wait()
        g_copy(i0 + 1, 1, gsem1).start()
        g_copy(i0 + 1, 1, gsem1).wait()
        s_copy(i0 + 1, 1, ssem1).start()
        s_copy(i0, 0, ssem0).wait()
        g_copy(i0 + 2, 0, gsem0).start()
        return carry

    lax.fori_loop(1, P - 1, pair, 0)

    i0 = N - 2
    g_copy(i0, 0, gsem0).wait()
    s_copy(i0, 0, ssem0).start()
    s_copy(i0 - 1, 1, ssem1).wait()
    g_copy(i0 + 1, 1, gsem1).start()
    g_copy(i0 + 1, 1, gsem1).wait()
    s_copy(i0 + 1, 1, ssem1).start()
    s_copy(i0, 0, ssem0).wait()
    s_copy(i0 + 1, 1, ssem1).wait()


@jax.jit
def _embed(sequence, weight):
    mesh = plsc.VectorSubcoreMesh(
        core_axis_name="c", subcore_axis_name="s",
        num_cores=NUM_CORES, num_subcores=NUM_SUBCORES)

    gather_k = pl.kernel(
        _body,
        out_type=jax.ShapeDtypeStruct((B, D), jnp.float32),
        mesh=mesh,
        scratch_types=[
            pltpu.VMEM((B_PER_W,), jnp.int32),
            pltpu.VMEM((2, CHUNK, D), jnp.float32),
            pltpu.SemaphoreType.DMA,
            pltpu.SemaphoreType.DMA,
            pltpu.SemaphoreType.DMA,
            pltpu.SemaphoreType.DMA,
        ],
        compiler_params=pltpu.CompilerParams(use_tc_tiling_on_sc=False),
    )
    seq_t = sequence.T.reshape(-1)
    out = gather_k(seq_t, weight)
    return jnp.swapaxes(out.reshape(SEQ_COLS, SEQ_ROWS, D), 0, 1)


def kernel(sequence, weight):
    return _embed(sequence, weight)
